# baseline (speedup 1.0000x reference)
import functools

import jax
import jax.numpy as jnp
from jax import lax
from jax.experimental import pallas as pl
from jax.experimental.pallas import tpu as pltpu
import jax.experimental.pallas.tpu_sc as plsc

B, T, D = 4, 2048, 768
E = 64
H = 2048
N = B * T
CAP = int(1.25 * N * 2 / E)
CAP2 = 2 * CAP
TB = 512
NB = N // TB
NW = 32
EPW = E // NW
ROWS_PW = EPW * CAP2
TPW = N // NW
GCH = 64
HB = 512
NH = H // HB


def _router_body(x_ref, gw_ref, rc0_ref, rc1_ref, k0_ref, k1_ref,
                 w0_ref, w1_ref, aux_ref, z_ref, runc0, runc1, mesum):
    b = pl.program_id(0)

    @pl.when(b == 0)
    def _():
        runc0[...] = jnp.zeros((1, E), jnp.float32)
        runc1[...] = jnp.zeros((1, E), jnp.float32)
        mesum[...] = jnp.zeros((1, E), jnp.float32)
        z_ref[0, 0] = 0.0

    x = x_ref[...]
    logits = jnp.dot(x, gw_ref[...], preferred_element_type=jnp.float32)
    m = jnp.max(logits, axis=1, keepdims=True)
    ex = jnp.exp(logits - m)
    gates = ex / jnp.sum(ex, axis=1, keepdims=True)

    colid = lax.broadcasted_iota(jnp.int32, (TB, E), 1)
    g1 = jnp.max(gates, axis=1, keepdims=True)
    a1 = jnp.min(jnp.where(gates == g1, colid, E), axis=1)
    masked = jnp.where(colid == a1[:, None], -jnp.inf, gates)
    g2 = jnp.max(masked, axis=1, keepdims=True)
    a2 = jnp.min(jnp.where(masked == g2, colid, E), axis=1)

    oh0 = (colid == a1[:, None]).astype(jnp.float32)
    oh1 = (colid == a2[:, None]).astype(jnp.float32)

    tril = (lax.broadcasted_iota(jnp.int32, (TB, TB), 0)
            >= lax.broadcasted_iota(jnp.int32, (TB, TB), 1)).astype(jnp.float32)
    c0 = jnp.dot(tril, oh0, preferred_element_type=jnp.float32)
    c1 = jnp.dot(tril, oh1, preferred_element_type=jnp.float32)

    pos0 = jnp.sum(oh0 * (runc0[...] + c0), axis=1) - 1.0
    pos1 = jnp.sum(oh1 * (runc1[...] + c1), axis=1) - 1.0
    keep0 = pos0 < CAP
    keep1 = pos1 < CAP
    rc0 = jnp.where(keep0, a1 * CAP2 + pos0.astype(jnp.int32), 0)
    rc1 = jnp.where(keep1, a2 * CAP2 + CAP + pos1.astype(jnp.int32), 0)

    rc0_ref[...] = rc0[:, None]
    rc1_ref[...] = rc1[:, None]
    k0_ref[...] = keep0.astype(jnp.int32)[:, None]
    k1_ref[...] = keep1.astype(jnp.int32)[:, None]
    w0_ref[...] = (g1[:, 0] * keep0.astype(jnp.float32))[:, None]
    w1_ref[...] = (g2[:, 0] * keep1.astype(jnp.float32))[:, None]

    runc0[...] = runc0[...] + jnp.sum(oh0, axis=0)[None, :]
    runc1[...] = runc1[...] + jnp.sum(oh1, axis=0)[None, :]
    mesum[...] = mesum[...] + jnp.sum(gates, axis=0)[None, :]
    z_ref[0, 0] = z_ref[0, 0] + jnp.sum(logits * logits)

    @pl.when(b == NB - 1)
    def _():
        aux_ref[0, 0] = (E / (N * float(N))) * jnp.sum(mesum[...] * runc0[...])
        z_ref[0, 0] = z_ref[0, 0] / (N * E)


def _router(xf, gate_w):
    return pl.pallas_call(
        _router_body,
        grid=(NB,),
        in_specs=[
            pl.BlockSpec((TB, D), lambda b: (b, 0)),
            pl.BlockSpec((D, E), lambda b: (0, 0)),
        ],
        out_specs=[
            pl.BlockSpec((TB, 1), lambda b: (b, 0)),
            pl.BlockSpec((TB, 1), lambda b: (b, 0)),
            pl.BlockSpec((TB, 1), lambda b: (b, 0)),
            pl.BlockSpec((TB, 1), lambda b: (b, 0)),
            pl.BlockSpec((TB, 1), lambda b: (b, 0)),
            pl.BlockSpec((TB, 1), lambda b: (b, 0)),
            pl.BlockSpec(memory_space=pltpu.SMEM),
            pl.BlockSpec(memory_space=pltpu.SMEM),
        ],
        out_shape=[
            jax.ShapeDtypeStruct((N, 1), jnp.int32),
            jax.ShapeDtypeStruct((N, 1), jnp.int32),
            jax.ShapeDtypeStruct((N, 1), jnp.int32),
            jax.ShapeDtypeStruct((N, 1), jnp.int32),
            jax.ShapeDtypeStruct((N, 1), jnp.float32),
            jax.ShapeDtypeStruct((N, 1), jnp.float32),
            jax.ShapeDtypeStruct((1, 1), jnp.float32),
            jax.ShapeDtypeStruct((1, 1), jnp.float32),
        ],
        scratch_shapes=[
            pltpu.VMEM((1, E), jnp.float32),
            pltpu.VMEM((1, E), jnp.float32),
            pltpu.VMEM((1, E), jnp.float32),
        ],
        interpret=False,
    )(xf, gate_w)


def _dispatch_body(x_hbm, rc0_hbm, rc1_hbm, k0_hbm, k1_hbm, xbuf_hbm,
                   list_v, rc_v, kp_v, rows_v, sem):
    wid = lax.axis_index("s") * 2 + lax.axis_index("c")
    my_base = wid * ROWS_PW

    def zero_body(i, _):
        list_v[pl.ds(i * 16, 16)] = jnp.zeros((16,), jnp.int32)
        return 0
    lax.fori_loop(0, ROWS_PW // 16, zero_body, 0)

    lanes = lax.iota(jnp.int32, 16)
    for rck_hbm, kpk_hbm in ((rc0_hbm, k0_hbm), (rc1_hbm, k1_hbm)):
        for c in range(4):
            pltpu.sync_copy(rck_hbm.at[pl.ds(c * 2048, 2048)], rc_v)
            pltpu.sync_copy(kpk_hbm.at[pl.ds(c * 2048, 2048)], kp_v)

            def scan_body(i, _):
                r = rc_v[pl.ds(i * 16, 16)]
                kp = kp_v[pl.ds(i * 16, 16)]
                tid = c * 2048 + i * 16 + lanes
                msk = (kp > 0) & (r >= my_base) & (r < my_base + ROWS_PW)
                plsc.store_scatter(list_v, [r - my_base], tid, mask=msk)
                return 0
            lax.fori_loop(0, 2048 // 16, scan_body, 0)

    for j in range(ROWS_PW // GCH):
        e_loc = j // (CAP2 // GCH)
        r0 = (j % (CAP2 // GCH)) * GCH
        idx_ref = list_v.at[pl.ds(j * GCH, GCH)]
        pltpu.async_copy(x_hbm.at[idx_ref], rows_v, sem).wait()
        pltpu.sync_copy(rows_v, xbuf_hbm.at[EPW * wid + e_loc].at[pl.ds(r0, GCH)])


def _dispatch(xf, rc0, rc1, k0, k1):
    mesh = plsc.VectorSubcoreMesh(core_axis_name="c", subcore_axis_name="s")
    fn = functools.partial(
        pl.kernel,
        out_type=jax.ShapeDtypeStruct((E, CAP2, D), jnp.float32),
        mesh=mesh,
        scratch_types=[
            pltpu.VMEM((ROWS_PW,), jnp.int32),
            pltpu.VMEM((2048,), jnp.int32),
            pltpu.VMEM((2048,), jnp.int32),
            pltpu.VMEM((GCH, D), jnp.float32),
            pltpu.SemaphoreType.DMA,
        ],
        compiler_params=pltpu.CompilerParams(needs_layout_passes=False),
    )(_dispatch_body)
    return fn(xf, rc0, rc1, k0, k1)


def _ffn_body(xb_ref, wg_ref, wu_ref, wd_ref, yb_ref):
    h = pl.program_id(1)
    x = xb_ref[0]
    g = jnp.dot(x, wg_ref[0], preferred_element_type=jnp.float32)
    u = jnp.dot(x, wu_ref[0], preferred_element_type=jnp.float32)
    hid = (g * (1.0 / (1.0 + jnp.exp(-g)))) * u
    y = jnp.dot(hid, wd_ref[0], preferred_element_type=jnp.float32)

    @pl.when(h == 0)
    def _():
        yb_ref[0] = y

    @pl.when(h != 0)
    def _():
        yb_ref[0] = yb_ref[0] + y


def _ffn(xbuf, w_gate, w_up, w_down):
    return pl.pallas_call(
        _ffn_body,
        grid=(E, NH),
        in_specs=[
            pl.BlockSpec((1, CAP2, D), lambda e, h: (e, 0, 0)),
            pl.BlockSpec((1, D, HB), lambda e, h: (e, 0, h)),
            pl.BlockSpec((1, D, HB), lambda e, h: (e, 0, h)),
            pl.BlockSpec((1, HB, D), lambda e, h: (e, h, 0)),
        ],
        out_specs=pl.BlockSpec((1, CAP2, D), lambda e, h: (e, 0, 0)),
        out_shape=jax.ShapeDtypeStruct((E, CAP2, D), jnp.float32),
        interpret=False,
    )(xbuf, w_gate, w_up, w_down)


def _combine_body(ybuf_hbm, rc0_hbm, rc1_hbm, y0_hbm, y1_hbm,
                  idx_v, rows_v, sem):
    wid = lax.axis_index("s") * 2 + lax.axis_index("c")
    base = wid * TPW
    for rck_hbm, yk_hbm in ((rc0_hbm, y0_hbm), (rc1_hbm, y1_hbm)):
        for c in range(TPW // GCH):
            pltpu.sync_copy(rck_hbm.at[pl.ds(base + c * GCH, GCH)], idx_v)
            pltpu.async_copy(ybuf_hbm.at[idx_v], rows_v, sem).wait()
            pltpu.sync_copy(rows_v, yk_hbm.at[pl.ds(base + c * GCH, GCH)])


def _combine(ybuf_flat, rc0, rc1):
    mesh = plsc.VectorSubcoreMesh(core_axis_name="c", subcore_axis_name="s")
    fn = functools.partial(
        pl.kernel,
        out_type=[
            jax.ShapeDtypeStruct((N, D), jnp.float32),
            jax.ShapeDtypeStruct((N, D), jnp.float32),
        ],
        mesh=mesh,
        scratch_types=[
            pltpu.VMEM((GCH,), jnp.int32),
            pltpu.VMEM((GCH, D), jnp.float32),
            pltpu.SemaphoreType.DMA,
        ],
        compiler_params=pltpu.CompilerParams(needs_layout_passes=False),
    )(_combine_body)
    return fn(ybuf_flat, rc0, rc1)


def _wsum_body(y0_ref, y1_ref, w0_ref, w1_ref, out_ref):
    out_ref[...] = y0_ref[...] * w0_ref[...] + y1_ref[...] * w1_ref[...]


def _wsum(y0, y1, w0, w1):
    return pl.pallas_call(
        _wsum_body,
        grid=(NB,),
        in_specs=[
            pl.BlockSpec((TB, D), lambda b: (b, 0)),
            pl.BlockSpec((TB, D), lambda b: (b, 0)),
            pl.BlockSpec((TB, 1), lambda b: (b, 0)),
            pl.BlockSpec((TB, 1), lambda b: (b, 0)),
        ],
        out_specs=pl.BlockSpec((TB, D), lambda b: (b, 0)),
        out_shape=jax.ShapeDtypeStruct((N, D), jnp.float32),
        interpret=False,
    )(y0, y1, w0, w1)


@jax.jit
def kernel(x, gate_w, w_gate, w_up, w_down):
    xf = x.reshape(N, D)
    rc0, rc1, k0, k1, w0, w1, aux, z = _router(xf, gate_w)
    rc0f = rc0.reshape(N)
    rc1f = rc1.reshape(N)
    xbuf = _dispatch(xf, rc0f, rc1f, k0.reshape(N), k1.reshape(N))
    ybuf = _ffn(xbuf, w_gate, w_up, w_down)
    y0, y1 = _combine(ybuf.reshape(E * CAP2, D), rc0f, rc1f)
    out = _wsum(y0, y1, w0, w1)
    return out.reshape(B, T, D), aux.reshape(()), z.reshape(())


# speedup vs baseline: 2.4521x; 2.4521x over previous
import functools

import jax
import jax.numpy as jnp
from jax import lax
from jax.experimental import pallas as pl
from jax.experimental.pallas import tpu as pltpu
import jax.experimental.pallas.tpu_sc as plsc

B, T, D = 4, 2048, 768
E = 64
H = 2048
N = B * T
CAP = int(1.25 * N * 2 / E)
CAP2 = 2 * CAP
CAPP = CAP2 + 8
TRASH = CAP2
TB = 512
NB = N // TB
NW = 32
TPW = N // NW
GCH = 64
NCH = TPW // GCH
HB = 512
NH = H // HB


def _router_body(x_ref, gw_ref, rd0_ref, rd1_ref,
                 w0_ref, w1_ref, aux_ref, z_ref, runc0, runc1, mesum):
    b = pl.program_id(0)

    @pl.when(b == 0)
    def _():
        runc0[...] = jnp.zeros((1, E), jnp.float32)
        runc1[...] = jnp.zeros((1, E), jnp.float32)
        mesum[...] = jnp.zeros((1, E), jnp.float32)
        z_ref[0, 0] = 0.0

    x = x_ref[...]
    logits = jnp.dot(x, gw_ref[...], preferred_element_type=jnp.float32)
    m = jnp.max(logits, axis=1, keepdims=True)
    ex = jnp.exp(logits - m)
    gates = ex / jnp.sum(ex, axis=1, keepdims=True)

    colid = lax.broadcasted_iota(jnp.int32, (TB, E), 1)
    g1 = jnp.max(gates, axis=1, keepdims=True)
    a1 = jnp.min(jnp.where(gates == g1, colid, E), axis=1)
    masked = jnp.where(colid == a1[:, None], -jnp.inf, gates)
    g2 = jnp.max(masked, axis=1, keepdims=True)
    a2 = jnp.min(jnp.where(masked == g2, colid, E), axis=1)

    oh0 = (colid == a1[:, None]).astype(jnp.float32)
    oh1 = (colid == a2[:, None]).astype(jnp.float32)

    tril = (lax.broadcasted_iota(jnp.int32, (TB, TB), 0)
            >= lax.broadcasted_iota(jnp.int32, (TB, TB), 1)).astype(jnp.float32)
    c0 = jnp.dot(tril, oh0, preferred_element_type=jnp.float32)
    c1 = jnp.dot(tril, oh1, preferred_element_type=jnp.float32)

    pos0 = jnp.sum(oh0 * (runc0[...] + c0), axis=1) - 1.0
    pos1 = jnp.sum(oh1 * (runc1[...] + c1), axis=1) - 1.0
    keep0 = pos0 < CAP
    keep1 = pos1 < CAP
    rd0 = a1 * CAPP + jnp.where(keep0, pos0.astype(jnp.int32), TRASH)
    rd1 = a2 * CAPP + jnp.where(keep1, CAP + pos1.astype(jnp.int32), TRASH)

    rd0_ref[...] = rd0[:, None]
    rd1_ref[...] = rd1[:, None]
    w0_ref[...] = (g1[:, 0] * keep0.astype(jnp.float32))[:, None]
    w1_ref[...] = (g2[:, 0] * keep1.astype(jnp.float32))[:, None]

    runc0[...] = runc0[...] + jnp.sum(oh0, axis=0)[None, :]
    runc1[...] = runc1[...] + jnp.sum(oh1, axis=0)[None, :]
    mesum[...] = mesum[...] + jnp.sum(gates, axis=0)[None, :]
    z_ref[0, 0] = z_ref[0, 0] + jnp.sum(logits * logits)

    @pl.when(b == NB - 1)
    def _():
        aux_ref[0, 0] = (E / (N * float(N))) * jnp.sum(mesum[...] * runc0[...])
        z_ref[0, 0] = z_ref[0, 0] / (N * E)


def _router(xf, gate_w):
    return pl.pallas_call(
        _router_body,
        grid=(NB,),
        in_specs=[
            pl.BlockSpec((TB, D), lambda b: (b, 0)),
            pl.BlockSpec((D, E), lambda b: (0, 0)),
        ],
        out_specs=[
            pl.BlockSpec((TB, 1), lambda b: (b, 0)),
            pl.BlockSpec((TB, 1), lambda b: (b, 0)),
            pl.BlockSpec((TB, 1), lambda b: (b, 0)),
            pl.BlockSpec((TB, 1), lambda b: (b, 0)),
            pl.BlockSpec(memory_space=pltpu.SMEM),
            pl.BlockSpec(memory_space=pltpu.SMEM),
        ],
        out_shape=[
            jax.ShapeDtypeStruct((N, 1), jnp.int32),
            jax.ShapeDtypeStruct((N, 1), jnp.int32),
            jax.ShapeDtypeStruct((N, 1), jnp.float32),
            jax.ShapeDtypeStruct((N, 1), jnp.float32),
            jax.ShapeDtypeStruct((1, 1), jnp.float32),
            jax.ShapeDtypeStruct((1, 1), jnp.float32),
        ],
        scratch_shapes=[
            pltpu.VMEM((1, E), jnp.float32),
            pltpu.VMEM((1, E), jnp.float32),
            pltpu.VMEM((1, E), jnp.float32),
        ],
        interpret=False,
    )(xf, gate_w)


def _dispatch_body(x_hbm, rdall_hbm, xbuf_hbm, idx_v, rows0_v, rows1_v,
                   sem_l, sem_s):
    wid = lax.axis_index("s") * 2 + lax.axis_index("c")
    base = wid * TPW
    pltpu.sync_copy(rdall_hbm.at[wid], idx_v)
    bufs = (rows0_v, rows1_v)
    ld = pltpu.async_copy(x_hbm.at[pl.ds(base, GCH)], rows0_v, sem_l)
    for c in range(NCH):
        ld.wait()
        if c < NCH - 1:
            ld = pltpu.async_copy(
                x_hbm.at[pl.ds(base + (c + 1) * GCH, GCH)],
                bufs[(c + 1) % 2], sem_l)
        s0 = pltpu.async_copy(bufs[c % 2], xbuf_hbm.at[idx_v.at[c]], sem_s)
        s1 = pltpu.async_copy(bufs[c % 2], xbuf_hbm.at[idx_v.at[NCH + c]],
                              sem_s)
        s0.wait()
        s1.wait()


def _dispatch(xf, rdall):
    mesh = plsc.VectorSubcoreMesh(core_axis_name="c", subcore_axis_name="s")
    fn = functools.partial(
        pl.kernel,
        out_type=jax.ShapeDtypeStruct((E * CAPP, D), jnp.float32),
        mesh=mesh,
        scratch_types=[
            pltpu.VMEM((2 * NCH, GCH), jnp.int32),
            pltpu.VMEM((GCH, D), jnp.float32),
            pltpu.VMEM((GCH, D), jnp.float32),
            pltpu.SemaphoreType.DMA,
            pltpu.SemaphoreType.DMA,
        ],
        compiler_params=pltpu.CompilerParams(needs_layout_passes=False),
    )(_dispatch_body)
    return fn(xf, rdall)


def _ffn_body(xb_ref, wg_ref, wu_ref, wd_ref, yb_ref):
    h = pl.program_id(1)
    x = xb_ref[0].astype(jnp.bfloat16)
    g = jnp.dot(x, wg_ref[0].astype(jnp.bfloat16),
                preferred_element_type=jnp.float32)
    u = jnp.dot(x, wu_ref[0].astype(jnp.bfloat16),
                preferred_element_type=jnp.float32)
    hid = (g * (1.0 / (1.0 + jnp.exp(-g)))) * u
    y = jnp.dot(hid.astype(jnp.bfloat16), wd_ref[0].astype(jnp.bfloat16),
                preferred_element_type=jnp.float32)

    @pl.when(h == 0)
    def _():
        yb_ref[0] = y

    @pl.when(h != 0)
    def _():
        yb_ref[0] = yb_ref[0] + y


def _ffn(xbuf, w_gate, w_up, w_down):
    return pl.pallas_call(
        _ffn_body,
        grid=(E, NH),
        in_specs=[
            pl.BlockSpec((1, CAPP, D), lambda e, h: (e, 0, 0)),
            pl.BlockSpec((1, D, HB), lambda e, h: (e, 0, h)),
            pl.BlockSpec((1, D, HB), lambda e, h: (e, 0, h)),
            pl.BlockSpec((1, HB, D), lambda e, h: (e, h, 0)),
        ],
        out_specs=pl.BlockSpec((1, CAPP, D), lambda e, h: (e, 0, 0)),
        out_shape=jax.ShapeDtypeStruct((E, CAPP, D), jnp.float32),
        interpret=False,
    )(xbuf, w_gate, w_up, w_down)


def _combine_body(ybuf_hbm, rd0_hbm, rd1_hbm, y0_hbm, y1_hbm,
                  idx_v, rows_v, sem):
    wid = lax.axis_index("s") * 2 + lax.axis_index("c")
    base = wid * TPW
    for rck_hbm, yk_hbm in ((rd0_hbm, y0_hbm), (rd1_hbm, y1_hbm)):
        for c in range(NCH):
            pltpu.sync_copy(rck_hbm.at[pl.ds(base + c * GCH, GCH)], idx_v)
            pltpu.async_copy(ybuf_hbm.at[idx_v], rows_v, sem).wait()
            pltpu.sync_copy(rows_v, yk_hbm.at[pl.ds(base + c * GCH, GCH)])


def _combine(ybuf_flat, rd0, rd1):
    mesh = plsc.VectorSubcoreMesh(core_axis_name="c", subcore_axis_name="s")
    fn = functools.partial(
        pl.kernel,
        out_type=[
            jax.ShapeDtypeStruct((N, D), jnp.float32),
            jax.ShapeDtypeStruct((N, D), jnp.float32),
        ],
        mesh=mesh,
        scratch_types=[
            pltpu.VMEM((GCH,), jnp.int32),
            pltpu.VMEM((GCH, D), jnp.float32),
            pltpu.SemaphoreType.DMA,
        ],
        compiler_params=pltpu.CompilerParams(needs_layout_passes=False),
    )(_combine_body)
    return fn(ybuf_flat, rd0, rd1)


def _wsum_body(y0_ref, y1_ref, w0_ref, w1_ref, out_ref):
    out_ref[...] = y0_ref[...] * w0_ref[...] + y1_ref[...] * w1_ref[...]


def _wsum(y0, y1, w0, w1):
    return pl.pallas_call(
        _wsum_body,
        grid=(NB,),
        in_specs=[
            pl.BlockSpec((TB, D), lambda b: (b, 0)),
            pl.BlockSpec((TB, D), lambda b: (b, 0)),
            pl.BlockSpec((TB, 1), lambda b: (b, 0)),
            pl.BlockSpec((TB, 1), lambda b: (b, 0)),
        ],
        out_specs=pl.BlockSpec((TB, D), lambda b: (b, 0)),
        out_shape=jax.ShapeDtypeStruct((N, D), jnp.float32),
        interpret=False,
    )(y0, y1, w0, w1)


@jax.jit
def kernel(x, gate_w, w_gate, w_up, w_down):
    xf = x.reshape(N, D)
    rd0, rd1, w0, w1, aux, z = _router(xf, gate_w)
    rd0f = rd0.reshape(N)
    rd1f = rd1.reshape(N)
    rdall = jnp.concatenate(
        [rd0.reshape(NW, NCH, GCH), rd1.reshape(NW, NCH, GCH)], axis=1)
    xbuf = _dispatch(xf, rdall)
    ybuf = _ffn(xbuf.reshape(E, CAPP, D), w_gate, w_up, w_down)
    y0, y1 = _combine(ybuf.reshape(E * CAPP, D), rd0f, rd1f)
    out = _wsum(y0, y1, w0, w1)
    return out.reshape(B, T, D), aux.reshape(()), z.reshape(())


# speedup vs baseline: 2.8800x; 1.1745x over previous
import functools

import jax
import jax.numpy as jnp
from jax import lax
from jax.experimental import pallas as pl
from jax.experimental.pallas import tpu as pltpu
import jax.experimental.pallas.tpu_sc as plsc

B, T, D = 4, 2048, 768
E = 64
H = 2048
N = B * T
CAP = int(1.25 * N * 2 / E)
CAP2 = 2 * CAP
CAPP = CAP2 + 8
TRASH = CAP2
TB = 512
NB = N // TB
NW = 32
TPW = N // NW
GCH = 64
NCH = TPW // GCH
CCH = 32
HB = 1024
NH = H // HB


def _router_body(x_ref, gw_ref, rd0_ref, rd1_ref,
                 w0_ref, w1_ref, aux_ref, z_ref, runc0, runc1, mesum):
    b = pl.program_id(0)

    @pl.when(b == 0)
    def _():
        runc0[...] = jnp.zeros((1, E), jnp.float32)
        runc1[...] = jnp.zeros((1, E), jnp.float32)
        mesum[...] = jnp.zeros((1, E), jnp.float32)
        z_ref[0, 0] = 0.0

    x = x_ref[...]
    logits = jnp.dot(x, gw_ref[...], preferred_element_type=jnp.float32)
    m = jnp.max(logits, axis=1, keepdims=True)
    ex = jnp.exp(logits - m)
    gates = ex / jnp.sum(ex, axis=1, keepdims=True)

    colid = lax.broadcasted_iota(jnp.int32, (TB, E), 1)
    g1 = jnp.max(gates, axis=1, keepdims=True)
    a1 = jnp.min(jnp.where(gates == g1, colid, E), axis=1)
    masked = jnp.where(colid == a1[:, None], -jnp.inf, gates)
    g2 = jnp.max(masked, axis=1, keepdims=True)
    a2 = jnp.min(jnp.where(masked == g2, colid, E), axis=1)

    oh0 = (colid == a1[:, None]).astype(jnp.float32)
    oh1 = (colid == a2[:, None]).astype(jnp.float32)

    tril = (lax.broadcasted_iota(jnp.int32, (TB, TB), 0)
            >= lax.broadcasted_iota(jnp.int32, (TB, TB), 1)).astype(jnp.float32)
    c0 = jnp.dot(tril, oh0, preferred_element_type=jnp.float32)
    c1 = jnp.dot(tril, oh1, preferred_element_type=jnp.float32)

    pos0 = jnp.sum(oh0 * (runc0[...] + c0), axis=1) - 1.0
    pos1 = jnp.sum(oh1 * (runc1[...] + c1), axis=1) - 1.0
    keep0 = pos0 < CAP
    keep1 = pos1 < CAP
    rd0 = a1 * CAPP + jnp.where(keep0, pos0.astype(jnp.int32), TRASH)
    rd1 = a2 * CAPP + jnp.where(keep1, CAP + pos1.astype(jnp.int32), TRASH)

    rd0_ref[...] = rd0[:, None]
    rd1_ref[...] = rd1[:, None]
    w0_ref[...] = (g1[:, 0] * keep0.astype(jnp.float32))[:, None]
    w1_ref[...] = (g2[:, 0] * keep1.astype(jnp.float32))[:, None]

    runc0[...] = runc0[...] + jnp.sum(oh0, axis=0)[None, :]
    runc1[...] = runc1[...] + jnp.sum(oh1, axis=0)[None, :]
    mesum[...] = mesum[...] + jnp.sum(gates, axis=0)[None, :]
    z_ref[0, 0] = z_ref[0, 0] + jnp.sum(logits * logits)

    @pl.when(b == NB - 1)
    def _():
        aux_ref[0, 0] = (E / (N * float(N))) * jnp.sum(mesum[...] * runc0[...])
        z_ref[0, 0] = z_ref[0, 0] / (N * E)


def _router(xf, gate_w):
    return pl.pallas_call(
        _router_body,
        grid=(NB,),
        in_specs=[
            pl.BlockSpec((TB, D), lambda b: (b, 0)),
            pl.BlockSpec((D, E), lambda b: (0, 0)),
        ],
        out_specs=[
            pl.BlockSpec((TB, 1), lambda b: (b, 0)),
            pl.BlockSpec((TB, 1), lambda b: (b, 0)),
            pl.BlockSpec((TB, 1), lambda b: (b, 0)),
            pl.BlockSpec((TB, 1), lambda b: (b, 0)),
            pl.BlockSpec(memory_space=pltpu.SMEM),
            pl.BlockSpec(memory_space=pltpu.SMEM),
        ],
        out_shape=[
            jax.ShapeDtypeStruct((N, 1), jnp.int32),
            jax.ShapeDtypeStruct((N, 1), jnp.int32),
            jax.ShapeDtypeStruct((N, 1), jnp.float32),
            jax.ShapeDtypeStruct((N, 1), jnp.float32),
            jax.ShapeDtypeStruct((1, 1), jnp.float32),
            jax.ShapeDtypeStruct((1, 1), jnp.float32),
        ],
        scratch_shapes=[
            pltpu.VMEM((1, E), jnp.float32),
            pltpu.VMEM((1, E), jnp.float32),
            pltpu.VMEM((1, E), jnp.float32),
        ],
        interpret=False,
    )(xf, gate_w)


def _dispatch_body(x_hbm, rdall_hbm, xbuf_hbm, idx_v, rows0_v, rows1_v,
                   sem_l, sem_s):
    wid = lax.axis_index("s") * 2 + lax.axis_index("c")
    base = wid * TPW
    pltpu.sync_copy(rdall_hbm.at[wid], idx_v)
    bufs = (rows0_v, rows1_v)
    ld = pltpu.async_copy(x_hbm.at[pl.ds(base, GCH)], rows0_v, sem_l)
    for c in range(NCH):
        ld.wait()
        if c < NCH - 1:
            ld = pltpu.async_copy(
                x_hbm.at[pl.ds(base + (c + 1) * GCH, GCH)],
                bufs[(c + 1) % 2], sem_l)
        s0 = pltpu.async_copy(bufs[c % 2], xbuf_hbm.at[idx_v.at[c]], sem_s)
        s1 = pltpu.async_copy(bufs[c % 2], xbuf_hbm.at[idx_v.at[NCH + c]],
                              sem_s)
        s0.wait()
        s1.wait()


def _dispatch(xf, rdall):
    mesh = plsc.VectorSubcoreMesh(core_axis_name="c", subcore_axis_name="s")
    fn = functools.partial(
        pl.kernel,
        out_type=jax.ShapeDtypeStruct((E * CAPP, D), jnp.float32),
        mesh=mesh,
        scratch_types=[
            pltpu.VMEM((2 * NCH, GCH), jnp.int32),
            pltpu.VMEM((GCH, D), jnp.float32),
            pltpu.VMEM((GCH, D), jnp.float32),
            pltpu.SemaphoreType.DMA,
            pltpu.SemaphoreType.DMA,
        ],
        compiler_params=pltpu.CompilerParams(needs_layout_passes=False),
    )(_dispatch_body)
    return fn(xf, rdall)


def _ffn_body(xb_ref, wg_ref, wu_ref, wd_ref, yb_ref):
    h = pl.program_id(1)
    x = xb_ref[0].astype(jnp.bfloat16)
    g = jnp.dot(x, wg_ref[0].astype(jnp.bfloat16),
                preferred_element_type=jnp.float32)
    u = jnp.dot(x, wu_ref[0].astype(jnp.bfloat16),
                preferred_element_type=jnp.float32)
    hid = (g * (1.0 / (1.0 + jnp.exp(-g)))) * u
    y = jnp.dot(hid.astype(jnp.bfloat16), wd_ref[0].astype(jnp.bfloat16),
                preferred_element_type=jnp.float32)

    @pl.when(h == 0)
    def _():
        yb_ref[0] = y

    @pl.when(h != 0)
    def _():
        yb_ref[0] = yb_ref[0] + y


def _ffn(xbuf, w_gate, w_up, w_down):
    return pl.pallas_call(
        _ffn_body,
        grid=(E, NH),
        in_specs=[
            pl.BlockSpec((1, CAPP, D), lambda e, h: (e, 0, 0)),
            pl.BlockSpec((1, D, HB), lambda e, h: (e, 0, h)),
            pl.BlockSpec((1, D, HB), lambda e, h: (e, 0, h)),
            pl.BlockSpec((1, HB, D), lambda e, h: (e, h, 0)),
        ],
        out_specs=pl.BlockSpec((1, CAPP, D), lambda e, h: (e, 0, 0)),
        out_shape=jax.ShapeDtypeStruct((E, CAPP, D), jnp.float32),
        interpret=False,
    )(xbuf, w_gate, w_up, w_down)


def _combine_body(ybuf_hbm, rd0_hbm, rd1_hbm, w0_hbm, w1_hbm, out_hbm,
                  idx0_v, idx1_v, w0_v, w1_v, rows0_v, rows1_v, out_v,
                  sem0, sem1, sem_o):
    wid = lax.axis_index("s") * 2 + lax.axis_index("c")
    base = wid * TPW
    pltpu.sync_copy(w0_hbm.at[pl.ds(base, TPW)], w0_v)
    pltpu.sync_copy(w1_hbm.at[pl.ds(base, TPW)], w1_v)
    st = None
    for c in range(TPW // CCH):
        pltpu.sync_copy(rd0_hbm.at[pl.ds(base + c * CCH, CCH)], idx0_v)
        pltpu.sync_copy(rd1_hbm.at[pl.ds(base + c * CCH, CCH)], idx1_v)
        g0 = pltpu.async_copy(ybuf_hbm.at[idx0_v], rows0_v, sem0)
        g1 = pltpu.async_copy(ybuf_hbm.at[idx1_v], rows1_v, sem1)
        g0.wait()
        g1.wait()
        if st is not None:
            st.wait()

        def row_body(i, _):
            lane_i = jnp.full((16,), c * CCH + i, jnp.int32)
            w0b = plsc.load_gather(w0_v, [lane_i])
            w1b = plsc.load_gather(w1_v, [lane_i])
            for d in range(D // 16):
                sl = pl.ds(d * 16, 16)
                out_v[i, sl] = rows0_v[i, sl] * w0b + rows1_v[i, sl] * w1b
            return 0
        lax.fori_loop(0, CCH, row_body, 0)
        st = pltpu.async_copy(out_v, out_hbm.at[pl.ds(base + c * CCH, CCH)],
                              sem_o)
    st.wait()


def _combine(ybuf_flat, rd0, rd1, w0, w1):
    mesh = plsc.VectorSubcoreMesh(core_axis_name="c", subcore_axis_name="s")
    fn = functools.partial(
        pl.kernel,
        out_type=jax.ShapeDtypeStruct((N, D), jnp.float32),
        mesh=mesh,
        scratch_types=[
            pltpu.VMEM((CCH,), jnp.int32),
            pltpu.VMEM((CCH,), jnp.int32),
            pltpu.VMEM((TPW,), jnp.float32),
            pltpu.VMEM((TPW,), jnp.float32),
            pltpu.VMEM((CCH, D), jnp.float32),
            pltpu.VMEM((CCH, D), jnp.float32),
            pltpu.VMEM((CCH, D), jnp.float32),
            pltpu.SemaphoreType.DMA,
            pltpu.SemaphoreType.DMA,
            pltpu.SemaphoreType.DMA,
        ],
        compiler_params=pltpu.CompilerParams(needs_layout_passes=False),
    )(_combine_body)
    return fn(ybuf_flat, rd0, rd1, w0, w1)


@jax.jit
def kernel(x, gate_w, w_gate, w_up, w_down):
    xf = x.reshape(N, D)
    rd0, rd1, w0, w1, aux, z = _router(xf, gate_w)
    rd0f = rd0.reshape(N)
    rd1f = rd1.reshape(N)
    rdall = jnp.concatenate(
        [rd0.reshape(NW, NCH, GCH), rd1.reshape(NW, NCH, GCH)], axis=1)
    xbuf = _dispatch(xf, rdall)
    ybuf = _ffn(xbuf.reshape(E, CAPP, D), w_gate, w_up, w_down)
    out = _combine(ybuf.reshape(E * CAPP, D), rd0f, rd1f,
                   w0.reshape(N), w1.reshape(N))
    return out.reshape(B, T, D), aux.reshape(()), z.reshape(())


# speedup vs baseline: 2.8940x; 1.0049x over previous
import functools

import jax
import jax.numpy as jnp
from jax import lax
from jax.experimental import pallas as pl
from jax.experimental.pallas import tpu as pltpu
import jax.experimental.pallas.tpu_sc as plsc

B, T, D = 4, 2048, 768
E = 64
H = 2048
N = B * T
CAP = int(1.25 * N * 2 / E)
CAP2 = 2 * CAP
CAPP = CAP2 + 8
TRASH = CAP2
TB = 512
NB = N // TB
NW = 32
TPW = N // NW
GCH = 64
NCH = TPW // GCH
CCH = 32
HB = 1024
NH = H // HB


def _router_body(x_ref, gw_ref, rd0_ref, rd1_ref,
                 w0_ref, w1_ref, aux_ref, z_ref, runc0, runc1, mesum):
    b = pl.program_id(0)

    @pl.when(b == 0)
    def _():
        runc0[...] = jnp.zeros((1, E), jnp.float32)
        runc1[...] = jnp.zeros((1, E), jnp.float32)
        mesum[...] = jnp.zeros((1, E), jnp.float32)
        z_ref[0, 0] = 0.0

    x = x_ref[...]
    logits = jnp.dot(x, gw_ref[...], preferred_element_type=jnp.float32)
    m = jnp.max(logits, axis=1, keepdims=True)
    ex = jnp.exp(logits - m)
    gates = ex / jnp.sum(ex, axis=1, keepdims=True)

    colid = lax.broadcasted_iota(jnp.int32, (TB, E), 1)
    g1 = jnp.max(gates, axis=1, keepdims=True)
    a1 = jnp.min(jnp.where(gates == g1, colid, E), axis=1)
    masked = jnp.where(colid == a1[:, None], -jnp.inf, gates)
    g2 = jnp.max(masked, axis=1, keepdims=True)
    a2 = jnp.min(jnp.where(masked == g2, colid, E), axis=1)

    oh0 = (colid == a1[:, None]).astype(jnp.float32)
    oh1 = (colid == a2[:, None]).astype(jnp.float32)

    tril = (lax.broadcasted_iota(jnp.int32, (TB, TB), 0)
            >= lax.broadcasted_iota(jnp.int32, (TB, TB), 1)).astype(jnp.float32)
    c0 = jnp.dot(tril, oh0, preferred_element_type=jnp.float32)
    c1 = jnp.dot(tril, oh1, preferred_element_type=jnp.float32)

    pos0 = jnp.sum(oh0 * (runc0[...] + c0), axis=1) - 1.0
    pos1 = jnp.sum(oh1 * (runc1[...] + c1), axis=1) - 1.0
    keep0 = pos0 < CAP
    keep1 = pos1 < CAP
    rd0 = a1 * CAPP + jnp.where(keep0, pos0.astype(jnp.int32), TRASH)
    rd1 = a2 * CAPP + jnp.where(keep1, CAP + pos1.astype(jnp.int32), TRASH)

    rd0_ref[...] = rd0[:, None]
    rd1_ref[...] = rd1[:, None]
    w0_ref[...] = (g1[:, 0] * keep0.astype(jnp.float32))[:, None]
    w1_ref[...] = (g2[:, 0] * keep1.astype(jnp.float32))[:, None]

    runc0[...] = runc0[...] + jnp.sum(oh0, axis=0)[None, :]
    runc1[...] = runc1[...] + jnp.sum(oh1, axis=0)[None, :]
    mesum[...] = mesum[...] + jnp.sum(gates, axis=0)[None, :]
    z_ref[0, 0] = z_ref[0, 0] + jnp.sum(logits * logits)

    @pl.when(b == NB - 1)
    def _():
        aux_ref[0, 0] = (E / (N * float(N))) * jnp.sum(mesum[...] * runc0[...])
        z_ref[0, 0] = z_ref[0, 0] / (N * E)


def _router(xf, gate_w):
    return pl.pallas_call(
        _router_body,
        grid=(NB,),
        in_specs=[
            pl.BlockSpec((TB, D), lambda b: (b, 0)),
            pl.BlockSpec((D, E), lambda b: (0, 0)),
        ],
        out_specs=[
            pl.BlockSpec((TB, 1), lambda b: (b, 0)),
            pl.BlockSpec((TB, 1), lambda b: (b, 0)),
            pl.BlockSpec((TB, 1), lambda b: (b, 0)),
            pl.BlockSpec((TB, 1), lambda b: (b, 0)),
            pl.BlockSpec(memory_space=pltpu.SMEM),
            pl.BlockSpec(memory_space=pltpu.SMEM),
        ],
        out_shape=[
            jax.ShapeDtypeStruct((N, 1), jnp.int32),
            jax.ShapeDtypeStruct((N, 1), jnp.int32),
            jax.ShapeDtypeStruct((N, 1), jnp.float32),
            jax.ShapeDtypeStruct((N, 1), jnp.float32),
            jax.ShapeDtypeStruct((1, 1), jnp.float32),
            jax.ShapeDtypeStruct((1, 1), jnp.float32),
        ],
        scratch_shapes=[
            pltpu.VMEM((1, E), jnp.float32),
            pltpu.VMEM((1, E), jnp.float32),
            pltpu.VMEM((1, E), jnp.float32),
        ],
        interpret=False,
    )(xf, gate_w)


def _dispatch_body(x_hbm, rdall_hbm, xbuf_hbm, idx_v, rows0_v, rows1_v,
                   sem_l, sem_s):
    wid = lax.axis_index("s") * 2 + lax.axis_index("c")
    base = wid * TPW
    pltpu.sync_copy(rdall_hbm.at[wid], idx_v)
    bufs = (rows0_v, rows1_v)
    ld = pltpu.async_copy(x_hbm.at[pl.ds(base, GCH)], rows0_v, sem_l)
    for c in range(NCH):
        ld.wait()
        if c < NCH - 1:
            ld = pltpu.async_copy(
                x_hbm.at[pl.ds(base + (c + 1) * GCH, GCH)],
                bufs[(c + 1) % 2], sem_l)
        s0 = pltpu.async_copy(bufs[c % 2], xbuf_hbm.at[idx_v.at[c]], sem_s)
        s1 = pltpu.async_copy(bufs[c % 2], xbuf_hbm.at[idx_v.at[NCH + c]],
                              sem_s)
        s0.wait()
        s1.wait()


def _dispatch(xf, rdall):
    mesh = plsc.VectorSubcoreMesh(core_axis_name="c", subcore_axis_name="s")
    fn = functools.partial(
        pl.kernel,
        out_type=jax.ShapeDtypeStruct((E * CAPP, D), jnp.float32),
        mesh=mesh,
        scratch_types=[
            pltpu.VMEM((2 * NCH, GCH), jnp.int32),
            pltpu.VMEM((GCH, D), jnp.float32),
            pltpu.VMEM((GCH, D), jnp.float32),
            pltpu.SemaphoreType.DMA,
            pltpu.SemaphoreType.DMA,
        ],
        compiler_params=pltpu.CompilerParams(needs_layout_passes=False),
    )(_dispatch_body)
    return fn(xf, rdall)


def _ffn_body(xb_ref, wg_ref, wu_ref, wd_ref, yb_ref):
    h = pl.program_id(1)
    x = xb_ref[0]
    g = jnp.dot(x, wg_ref[0], preferred_element_type=jnp.float32)
    u = jnp.dot(x, wu_ref[0], preferred_element_type=jnp.float32)
    hid = (g * (1.0 / (1.0 + jnp.exp(-g)))) * u
    y = jnp.dot(hid, wd_ref[0], preferred_element_type=jnp.float32)

    @pl.when(h == 0)
    def _():
        yb_ref[0] = y

    @pl.when(h != 0)
    def _():
        yb_ref[0] = yb_ref[0] + y


def _ffn(xbuf, w_gate, w_up, w_down):
    return pl.pallas_call(
        _ffn_body,
        grid=(E, NH),
        in_specs=[
            pl.BlockSpec((1, CAPP, D), lambda e, h: (e, 0, 0)),
            pl.BlockSpec((1, D, HB), lambda e, h: (e, 0, h)),
            pl.BlockSpec((1, D, HB), lambda e, h: (e, 0, h)),
            pl.BlockSpec((1, HB, D), lambda e, h: (e, h, 0)),
        ],
        out_specs=pl.BlockSpec((1, CAPP, D), lambda e, h: (e, 0, 0)),
        out_shape=jax.ShapeDtypeStruct((E, CAPP, D), jnp.float32),
        interpret=False,
    )(xbuf, w_gate, w_up, w_down)


def _combine_body(ybuf_hbm, rd0_hbm, rd1_hbm, w0_hbm, w1_hbm, out_hbm,
                  idx0_v, idx1_v, w0_v, w1_v, rows0_v, rows1_v, out_v,
                  sem0, sem1, sem_o):
    wid = lax.axis_index("s") * 2 + lax.axis_index("c")
    base = wid * TPW
    pltpu.sync_copy(w0_hbm.at[pl.ds(base, TPW)], w0_v)
    pltpu.sync_copy(w1_hbm.at[pl.ds(base, TPW)], w1_v)
    st = None
    for c in range(TPW // CCH):
        pltpu.sync_copy(rd0_hbm.at[pl.ds(base + c * CCH, CCH)], idx0_v)
        pltpu.sync_copy(rd1_hbm.at[pl.ds(base + c * CCH, CCH)], idx1_v)
        g0 = pltpu.async_copy(ybuf_hbm.at[idx0_v], rows0_v, sem0)
        g1 = pltpu.async_copy(ybuf_hbm.at[idx1_v], rows1_v, sem1)
        g0.wait()
        g1.wait()
        if st is not None:
            st.wait()

        def row_body(i, _):
            lane_i = jnp.full((16,), c * CCH + i, jnp.int32)
            w0b = plsc.load_gather(w0_v, [lane_i])
            w1b = plsc.load_gather(w1_v, [lane_i])
            for d in range(D // 16):
                sl = pl.ds(d * 16, 16)
                out_v[i, sl] = rows0_v[i, sl] * w0b + rows1_v[i, sl] * w1b
            return 0
        lax.fori_loop(0, CCH, row_body, 0)
        st = pltpu.async_copy(out_v, out_hbm.at[pl.ds(base + c * CCH, CCH)],
                              sem_o)
    st.wait()


def _combine(ybuf_flat, rd0, rd1, w0, w1):
    mesh = plsc.VectorSubcoreMesh(core_axis_name="c", subcore_axis_name="s")
    fn = functools.partial(
        pl.kernel,
        out_type=jax.ShapeDtypeStruct((N, D), jnp.float32),
        mesh=mesh,
        scratch_types=[
            pltpu.VMEM((CCH,), jnp.int32),
            pltpu.VMEM((CCH,), jnp.int32),
            pltpu.VMEM((TPW,), jnp.float32),
            pltpu.VMEM((TPW,), jnp.float32),
            pltpu.VMEM((CCH, D), jnp.float32),
            pltpu.VMEM((CCH, D), jnp.float32),
            pltpu.VMEM((CCH, D), jnp.float32),
            pltpu.SemaphoreType.DMA,
            pltpu.SemaphoreType.DMA,
            pltpu.SemaphoreType.DMA,
        ],
        compiler_params=pltpu.CompilerParams(needs_layout_passes=False),
    )(_combine_body)
    return fn(ybuf_flat, rd0, rd1, w0, w1)


@jax.jit
def kernel(x, gate_w, w_gate, w_up, w_down):
    xf = x.reshape(N, D)
    rd0, rd1, w0, w1, aux, z = _router(xf, gate_w)
    rd0f = rd0.reshape(N)
    rd1f = rd1.reshape(N)
    rdall = jnp.concatenate(
        [rd0.reshape(NW, NCH, GCH), rd1.reshape(NW, NCH, GCH)], axis=1)
    xbuf = _dispatch(xf, rdall)
    ybuf = _ffn(xbuf.reshape(E, CAPP, D), w_gate, w_up, w_down)
    out = _combine(ybuf.reshape(E * CAPP, D), rd0f, rd1f,
                   w0.reshape(N), w1.reshape(N))
    return out.reshape(B, T, D), aux.reshape(()), z.reshape(())


# speedup vs baseline: 2.9768x; 1.0286x over previous
import functools

import jax
import jax.numpy as jnp
from jax import lax
from jax.experimental import pallas as pl
from jax.experimental.pallas import tpu as pltpu
import jax.experimental.pallas.tpu_sc as plsc

B, T, D = 4, 2048, 768
E = 64
H = 2048
N = B * T
CAP = int(1.25 * N * 2 / E)
CAP2 = 2 * CAP
CAPP = CAP2 + 8
TRASH = CAP2
TB = 512
NB = N // TB
NW = 32
TPW = N // NW
GCH = 64
NCH = TPW // GCH
CCH = 32
HB = 1024
NH = H // HB


def _router_body(x_ref, gw_ref, rd0_ref, rd1_ref,
                 w0_ref, w1_ref, aux_ref, z_ref, runc0, runc1, mesum):
    b = pl.program_id(0)

    @pl.when(b == 0)
    def _():
        runc0[...] = jnp.zeros((1, E), jnp.float32)
        runc1[...] = jnp.zeros((1, E), jnp.float32)
        mesum[...] = jnp.zeros((1, E), jnp.float32)
        z_ref[0, 0] = 0.0

    x = x_ref[...]
    logits = jnp.dot(x, gw_ref[...], preferred_element_type=jnp.float32)
    m = jnp.max(logits, axis=1, keepdims=True)
    ex = jnp.exp(logits - m)
    gates = ex / jnp.sum(ex, axis=1, keepdims=True)

    colid = lax.broadcasted_iota(jnp.int32, (TB, E), 1)
    g1 = jnp.max(gates, axis=1, keepdims=True)
    a1 = jnp.min(jnp.where(gates == g1, colid, E), axis=1)
    masked = jnp.where(colid == a1[:, None], -jnp.inf, gates)
    g2 = jnp.max(masked, axis=1, keepdims=True)
    a2 = jnp.min(jnp.where(masked == g2, colid, E), axis=1)

    oh0 = (colid == a1[:, None]).astype(jnp.float32)
    oh1 = (colid == a2[:, None]).astype(jnp.float32)

    tril = (lax.broadcasted_iota(jnp.int32, (TB, TB), 0)
            >= lax.broadcasted_iota(jnp.int32, (TB, TB), 1)).astype(jnp.float32)
    c0 = jnp.dot(tril, oh0, preferred_element_type=jnp.float32)
    c1 = jnp.dot(tril, oh1, preferred_element_type=jnp.float32)

    pos0 = jnp.sum(oh0 * (runc0[...] + c0), axis=1) - 1.0
    pos1 = jnp.sum(oh1 * (runc1[...] + c1), axis=1) - 1.0
    keep0 = pos0 < CAP
    keep1 = pos1 < CAP
    rd0 = a1 * CAPP + jnp.where(keep0, pos0.astype(jnp.int32), TRASH)
    rd1 = a2 * CAPP + jnp.where(keep1, CAP + pos1.astype(jnp.int32), TRASH)

    rd0_ref[...] = rd0[:, None]
    rd1_ref[...] = rd1[:, None]
    w0_ref[...] = (g1[:, 0] * keep0.astype(jnp.float32))[:, None]
    w1_ref[...] = (g2[:, 0] * keep1.astype(jnp.float32))[:, None]

    runc0[...] = runc0[...] + jnp.sum(oh0, axis=0)[None, :]
    runc1[...] = runc1[...] + jnp.sum(oh1, axis=0)[None, :]
    mesum[...] = mesum[...] + jnp.sum(gates, axis=0)[None, :]
    z_ref[0, 0] = z_ref[0, 0] + jnp.sum(logits * logits)

    @pl.when(b == NB - 1)
    def _():
        aux_ref[0, 0] = (E / (N * float(N))) * jnp.sum(mesum[...] * runc0[...])
        z_ref[0, 0] = z_ref[0, 0] / (N * E)


def _router(xf, gate_w):
    return pl.pallas_call(
        _router_body,
        grid=(NB,),
        in_specs=[
            pl.BlockSpec((TB, D), lambda b: (b, 0)),
            pl.BlockSpec((D, E), lambda b: (0, 0)),
        ],
        out_specs=[
            pl.BlockSpec((TB, 1), lambda b: (b, 0)),
            pl.BlockSpec((TB, 1), lambda b: (b, 0)),
            pl.BlockSpec((TB, 1), lambda b: (b, 0)),
            pl.BlockSpec((TB, 1), lambda b: (b, 0)),
            pl.BlockSpec(memory_space=pltpu.SMEM),
            pl.BlockSpec(memory_space=pltpu.SMEM),
        ],
        out_shape=[
            jax.ShapeDtypeStruct((N, 1), jnp.int32),
            jax.ShapeDtypeStruct((N, 1), jnp.int32),
            jax.ShapeDtypeStruct((N, 1), jnp.float32),
            jax.ShapeDtypeStruct((N, 1), jnp.float32),
            jax.ShapeDtypeStruct((1, 1), jnp.float32),
            jax.ShapeDtypeStruct((1, 1), jnp.float32),
        ],
        scratch_shapes=[
            pltpu.VMEM((1, E), jnp.float32),
            pltpu.VMEM((1, E), jnp.float32),
            pltpu.VMEM((1, E), jnp.float32),
        ],
        interpret=False,
    )(xf, gate_w)


def _dispatch_body(x_hbm, rdall_hbm, xbuf_hbm, idx_v, rows0_v, rows1_v,
                   sem_l, sem_s):
    wid = lax.axis_index("s") * 2 + lax.axis_index("c")
    base = wid * TPW
    pltpu.sync_copy(rdall_hbm.at[wid], idx_v)
    bufs = (rows0_v, rows1_v)
    ld = pltpu.async_copy(x_hbm.at[pl.ds(base, GCH)], rows0_v, sem_l)
    for c in range(NCH):
        ld.wait()
        if c < NCH - 1:
            ld = pltpu.async_copy(
                x_hbm.at[pl.ds(base + (c + 1) * GCH, GCH)],
                bufs[(c + 1) % 2], sem_l)
        s0 = pltpu.async_copy(bufs[c % 2], xbuf_hbm.at[idx_v.at[c]], sem_s)
        s1 = pltpu.async_copy(bufs[c % 2], xbuf_hbm.at[idx_v.at[NCH + c]],
                              sem_s)
        s0.wait()
        s1.wait()


def _dispatch(xf, rdall):
    mesh = plsc.VectorSubcoreMesh(core_axis_name="c", subcore_axis_name="s")
    fn = functools.partial(
        pl.kernel,
        out_type=jax.ShapeDtypeStruct((E * CAPP, D), jnp.float32),
        mesh=mesh,
        scratch_types=[
            pltpu.VMEM((2 * NCH, GCH), jnp.int32),
            pltpu.VMEM((GCH, D), jnp.float32),
            pltpu.VMEM((GCH, D), jnp.float32),
            pltpu.SemaphoreType.DMA,
            pltpu.SemaphoreType.DMA,
        ],
        compiler_params=pltpu.CompilerParams(needs_layout_passes=False),
    )(_dispatch_body)
    return fn(xf, rdall)


def _ffn_body(xb_ref, wg_ref, wu_ref, wd_ref, yb_ref):
    h = pl.program_id(1)
    x = xb_ref[0]
    g = jnp.dot(x, wg_ref[0], preferred_element_type=jnp.float32)
    u = jnp.dot(x, wu_ref[0], preferred_element_type=jnp.float32)
    hid = (g * (1.0 / (1.0 + jnp.exp(-g)))) * u
    y = jnp.dot(hid, wd_ref[0], preferred_element_type=jnp.float32)

    @pl.when(h == 0)
    def _():
        yb_ref[0] = y

    @pl.when(h != 0)
    def _():
        yb_ref[0] = yb_ref[0] + y


def _ffn(xbuf, w_gate, w_up, w_down):
    return pl.pallas_call(
        _ffn_body,
        grid=(E, NH),
        in_specs=[
            pl.BlockSpec((1, CAPP, D), lambda e, h: (e, 0, 0)),
            pl.BlockSpec((1, D, HB), lambda e, h: (e, 0, h)),
            pl.BlockSpec((1, D, HB), lambda e, h: (e, 0, h)),
            pl.BlockSpec((1, HB, D), lambda e, h: (e, h, 0)),
        ],
        out_specs=pl.BlockSpec((1, CAPP, D), lambda e, h: (e, 0, 0)),
        out_shape=jax.ShapeDtypeStruct((E, CAPP, D), jnp.float32),
        interpret=False,
    )(xbuf, w_gate, w_up, w_down)


def _combine_body(ybuf_hbm, rd0_hbm, rd1_hbm, w0_hbm, w1_hbm, out_hbm,
                  idx00, idx01, idx10, idx11, w0_v, w1_v,
                  r0a, r0b, r1a, r1b, out_v, sem0, sem1, sem_o):
    wid = lax.axis_index("s") * 2 + lax.axis_index("c")
    base = wid * TPW
    pltpu.sync_copy(w0_hbm.at[pl.ds(base, TPW)], w0_v)
    pltpu.sync_copy(w1_hbm.at[pl.ds(base, TPW)], w1_v)
    idx0 = (idx00, idx01)
    idx1 = (idx10, idx11)
    r0 = (r0a, r0b)
    r1 = (r1a, r1b)
    nc = TPW // CCH
    pltpu.sync_copy(rd0_hbm.at[pl.ds(base, CCH)], idx00)
    pltpu.sync_copy(rd1_hbm.at[pl.ds(base, CCH)], idx10)
    g0 = pltpu.async_copy(ybuf_hbm.at[idx00], r0a, sem0)
    g1 = pltpu.async_copy(ybuf_hbm.at[idx10], r1a, sem1)
    st = None
    for c in range(nc):
        p = c % 2
        q = (c + 1) % 2
        if c < nc - 1:
            pltpu.sync_copy(rd0_hbm.at[pl.ds(base + (c + 1) * CCH, CCH)],
                            idx0[q])
            pltpu.sync_copy(rd1_hbm.at[pl.ds(base + (c + 1) * CCH, CCH)],
                            idx1[q])
        g0.wait()
        g1.wait()
        if c < nc - 1:
            g0 = pltpu.async_copy(ybuf_hbm.at[idx0[q]], r0[q], sem0)
            g1 = pltpu.async_copy(ybuf_hbm.at[idx1[q]], r1[q], sem1)
        if st is not None:
            st.wait()
        rp = r0[p]
        rq = r1[p]

        def row_body(i, _):
            lane_i = jnp.full((16,), c * CCH + i, jnp.int32)
            w0b = plsc.load_gather(w0_v, [lane_i])
            w1b = plsc.load_gather(w1_v, [lane_i])
            for d in range(D // 16):
                sl = pl.ds(d * 16, 16)
                out_v[i, sl] = rp[i, sl] * w0b + rq[i, sl] * w1b
            return 0
        lax.fori_loop(0, CCH, row_body, 0)
        st = pltpu.async_copy(out_v, out_hbm.at[pl.ds(base + c * CCH, CCH)],
                              sem_o)
    st.wait()


def _combine(ybuf_flat, rd0, rd1, w0, w1):
    mesh = plsc.VectorSubcoreMesh(core_axis_name="c", subcore_axis_name="s")
    fn = functools.partial(
        pl.kernel,
        out_type=jax.ShapeDtypeStruct((N, D), jnp.float32),
        mesh=mesh,
        scratch_types=[
            pltpu.VMEM((CCH,), jnp.int32),
            pltpu.VMEM((CCH,), jnp.int32),
            pltpu.VMEM((CCH,), jnp.int32),
            pltpu.VMEM((CCH,), jnp.int32),
            pltpu.VMEM((TPW,), jnp.float32),
            pltpu.VMEM((TPW,), jnp.float32),
            pltpu.VMEM((CCH, D), jnp.float32),
            pltpu.VMEM((CCH, D), jnp.float32),
            pltpu.VMEM((CCH, D), jnp.float32),
            pltpu.VMEM((CCH, D), jnp.float32),
            pltpu.VMEM((CCH, D), jnp.float32),
            pltpu.SemaphoreType.DMA,
            pltpu.SemaphoreType.DMA,
            pltpu.SemaphoreType.DMA,
        ],
        compiler_params=pltpu.CompilerParams(needs_layout_passes=False),
    )(_combine_body)
    return fn(ybuf_flat, rd0, rd1, w0, w1)


@jax.jit
def kernel(x, gate_w, w_gate, w_up, w_down):
    xf = x.reshape(N, D)
    rd0, rd1, w0, w1, aux, z = _router(xf, gate_w)
    rd0f = rd0.reshape(N)
    rd1f = rd1.reshape(N)
    rdall = jnp.concatenate(
        [rd0.reshape(NW, NCH, GCH), rd1.reshape(NW, NCH, GCH)], axis=1)
    xbuf = _dispatch(xf, rdall)
    ybuf = _ffn(xbuf.reshape(E, CAPP, D), w_gate, w_up, w_down)
    out = _combine(ybuf.reshape(E * CAPP, D), rd0f, rd1f,
                   w0.reshape(N), w1.reshape(N))
    return out.reshape(B, T, D), aux.reshape(()), z.reshape(())
